# Initial kernel scaffold; baseline (speedup 1.0000x reference)
import jax
import jax.numpy as jnp
from jax.experimental import pallas as pl


def kernel(x, codebooks):
    raise NotImplementedError("write your pallas kernel here")


# speedup vs baseline: 1.1357x; 1.1357x over previous
import jax
import jax.numpy as jnp
from jax.experimental import pallas as pl

NUM_STAGES = 8
K = 1024
D = 256
ROWS_PER_BLOCK = 512


def _rvq_kernel(h_ref, cb_ref, out_ref, q1_ref, q2_ref, commit_ref):
    first = pl.program_id(0) == 0
    r = h_ref[...]
    acc = jnp.zeros_like(r)
    for i in range(NUM_STAGES):
        cb = cb_ref[i]
        s = jax.lax.dot_general(r, cb, (((1,), (1,)), ((), ())),
                                preferred_element_type=jnp.float32)
        a = jnp.sum(r * r, axis=1, keepdims=True)
        cn = jnp.sum(cb * cb, axis=1)[None, :]
        d = (a - 2.0 * s) + cn
        dmin = jnp.min(d, axis=1, keepdims=True)
        ids = jax.lax.broadcasted_iota(jnp.int32, d.shape, 1)
        idx = jnp.min(jnp.where(d == dmin, ids, K), axis=1, keepdims=True)
        oh = (ids == idx).astype(jnp.float32)
        dn = (((1,), (0,)), ((), ()))
        e = jax.lax.dot_general(oh, cb, dn,
                                precision=jax.lax.Precision.HIGHEST,
                                preferred_element_type=jnp.float32)
        q = r + (e - r)
        csum = jnp.sum(dmin)
        row = jnp.full((1, 128), csum, jnp.float32)
        prev = jnp.where(first, jnp.zeros_like(row), commit_ref[i:i + 1, :])
        commit_ref[i:i + 1, :] = prev + row
        acc = acc + q
        if i == 0:
            q1_ref[...] = q
        if i == 1:
            q2_ref[...] = q
        r = r - q
    out_ref[...] = acc


def kernel(x, codebooks):
    B, Dx, T = x.shape
    n_rows = B * T
    h = jnp.transpose(x, (0, 2, 1)).reshape(n_rows, D)

    grid = (n_rows // ROWS_PER_BLOCK,)
    row_spec = pl.BlockSpec((ROWS_PER_BLOCK, D), lambda c: (c, 0))
    cb_spec = pl.BlockSpec((NUM_STAGES, K, D), lambda c: (0, 0, 0))
    out, q1, q2, commit = pl.pallas_call(
        _rvq_kernel,
        grid=grid,
        in_specs=[row_spec, cb_spec],
        out_specs=[row_spec, row_spec, row_spec,
                   pl.BlockSpec((NUM_STAGES, 128), lambda c: (0, 0))],
        out_shape=[
            jax.ShapeDtypeStruct((n_rows, D), jnp.float32),
            jax.ShapeDtypeStruct((n_rows, D), jnp.float32),
            jax.ShapeDtypeStruct((n_rows, D), jnp.float32),
            jax.ShapeDtypeStruct((NUM_STAGES, 128), jnp.float32),
        ],
    )(h, codebooks)

    def back(y):
        return jnp.transpose(y.reshape(B, T, Dx), (0, 2, 1))

    commits = commit[:, 0] / jnp.float32(n_rows * D)
    com = jnp.mean(commits)
    return (back(out), back(q1), back(q2), com)


# speedup vs baseline: 1.5410x; 1.3569x over previous
import jax
import jax.numpy as jnp
from jax.experimental import pallas as pl

NUM_STAGES = 8
K = 1024
D = 256
ROWS_PER_BLOCK = 512


def _rvq_kernel(h_ref, cb_ref, cb3_ref, out_ref, q1_ref, q2_ref, commit_ref):
    first = pl.program_id(0) == 0
    r = h_ref[...]
    acc = jnp.zeros_like(r)
    for i in range(NUM_STAGES):
        cb = cb_ref[i]
        s = jax.lax.dot_general(r, cb, (((1,), (1,)), ((), ())),
                                preferred_element_type=jnp.float32)
        a = jnp.sum(r * r, axis=1, keepdims=True)
        cn = jnp.sum(cb * cb, axis=1)[None, :]
        d = (a - 2.0 * s) + cn
        dmin = jnp.min(d, axis=1, keepdims=True)
        ids = jax.lax.broadcasted_iota(jnp.int32, d.shape, 1)
        idx = jnp.min(jnp.where(d == dmin, ids, K), axis=1, keepdims=True)
        ids3 = jax.lax.broadcasted_iota(jnp.int32, (r.shape[0], 3 * K), 1)
        oh3 = ((ids3 & (K - 1)) == idx).astype(jnp.bfloat16)
        dn = (((1,), (0,)), ((), ()))
        e = jax.lax.dot_general(oh3, cb3_ref[i], dn,
                                preferred_element_type=jnp.float32)
        q = r + (e - r)
        csum = jnp.sum(dmin)
        row = jnp.full((1, 128), csum, jnp.float32)
        prev = jnp.where(first, jnp.zeros_like(row), commit_ref[i:i + 1, :])
        commit_ref[i:i + 1, :] = prev + row
        acc = acc + q
        if i == 0:
            q1_ref[...] = q
        if i == 1:
            q2_ref[...] = q
        r = r - q
    out_ref[...] = acc


def kernel(x, codebooks):
    B, Dx, T = x.shape
    n_rows = B * T
    h = jnp.transpose(x, (0, 2, 1)).reshape(n_rows, D)

    mask = jnp.int32(-65536)
    def trunc_bf16(v):
        b = jax.lax.bitcast_convert_type(v, jnp.int32)
        return jax.lax.bitcast_convert_type(b & mask, jnp.float32)
    cbh = trunc_bf16(codebooks)
    rem = codebooks - cbh
    cbm = trunc_bf16(rem)
    cbl = rem - cbm
    cb3 = jnp.concatenate([cbh.astype(jnp.bfloat16),
                           cbm.astype(jnp.bfloat16),
                           cbl.astype(jnp.bfloat16)], axis=1)

    grid = (n_rows // ROWS_PER_BLOCK,)
    row_spec = pl.BlockSpec((ROWS_PER_BLOCK, D), lambda c: (c, 0))
    cb_spec = pl.BlockSpec((NUM_STAGES, K, D), lambda c: (0, 0, 0))
    cb3_spec = pl.BlockSpec((NUM_STAGES, 3 * K, D), lambda c: (0, 0, 0))
    out, q1, q2, commit = pl.pallas_call(
        _rvq_kernel,
        grid=grid,
        in_specs=[row_spec, cb_spec, cb3_spec],
        out_specs=[row_spec, row_spec, row_spec,
                   pl.BlockSpec((NUM_STAGES, 128), lambda c: (0, 0))],
        out_shape=[
            jax.ShapeDtypeStruct((n_rows, D), jnp.float32),
            jax.ShapeDtypeStruct((n_rows, D), jnp.float32),
            jax.ShapeDtypeStruct((n_rows, D), jnp.float32),
            jax.ShapeDtypeStruct((NUM_STAGES, 128), jnp.float32),
        ],
    )(h, codebooks, cb3)

    def back(y):
        return jnp.transpose(y.reshape(B, T, Dx), (0, 2, 1))

    commits = commit[:, 0] / jnp.float32(n_rows * D)
    com = jnp.mean(commits)
    return (back(out), back(q1), back(q2), com)
